# baseline (speedup 1.0000x reference)
import functools

import jax
import jax.numpy as jnp
from jax import lax
from jax.experimental import pallas as pl
from jax.experimental.pallas import tpu as pltpu
from jax.experimental.pallas import tpu_sc as plsc

_DIM = 33
_CSTRIDE = _DIM ** 3
_CPAD = 36096
_PLANE = 512 * 512
_B = 16
_PPW = _B * _PLANE // 32
_CHUNK = 1024
_NCHUNK = _PPW // _CHUNK
_GROUPS = _CHUNK // 16
_SCALE = float(_DIM - 1)


def _interp_group(luts, xin, xout, o):
    rr = o >> 7
    col = o & 127
    xr = xin[0, rr, pl.ds(col, 16)]
    xg = xin[1, rr, pl.ds(col, 16)]
    xb = xin[2, rr, pl.ds(col, 16)]
    r = xr * _SCALE
    g = xg * _SCALE
    b = xb * _SCALE
    r0 = jnp.minimum(r.astype(jnp.int32), _DIM - 2)
    g0 = jnp.minimum(g.astype(jnp.int32), _DIM - 2)
    b0 = jnp.minimum(b.astype(jnp.int32), _DIM - 2)
    fr = r - r0.astype(jnp.float32)
    fg = g - g0.astype(jnp.float32)
    fb = b - b0.astype(jnp.float32)
    base = b0 * (_DIM * _DIM) + g0 * _DIM + r0
    cg = 1.0 - fg
    cb = 1.0 - fb
    t00 = cb * cg
    t01 = cb * fg
    t10 = fb * cg
    t11 = fb * fg
    cr = 1.0 - fr
    acc0 = [None, None, None]
    acc1 = [None, None, None]
    for k, (off, tw, fw) in enumerate((
        (0, t00, cr), (1, t00, fr),
        (_DIM, t01, cr), (_DIM + 1, t01, fr),
        (_DIM * _DIM, t10, cr), (_DIM * _DIM + 1, t10, fr),
        (_DIM * _DIM + _DIM, t11, cr), (_DIM * _DIM + _DIM + 1, t11, fr),
    )):
        idx = base + off if off else base
        w = tw * fw
        acc = acc0 if k % 2 == 0 else acc1
        for c in range(3):
            v = w * plsc.load_gather(luts[c], [idx])
            acc[c] = v if acc[c] is None else acc[c] + v
    for c in range(3):
        xout[c, rr, pl.ds(col, 16)] = acc0[c] + acc1[c]


def _body(lut_hbm, x_hbm, out_hbm, lutr, lutg, lutb, xin0, xin1, xout0, xout1,
          si0, si1, so0, so1):
    img = lax.axis_index("s")
    half = lax.axis_index("c")
    base = half * 32
    luts = (lutr, lutg, lutb)
    lut_copies = [
        pltpu.make_async_copy(
            lut_hbm.at[pl.ds(c * _CPAD, _CSTRIDE)], luts[c], so0
        )
        for c in range(3)
    ]
    for d in lut_copies:
        d.start()

    def in_copy(g, buf, sem):
        tr = base + (g >> 2)
        tc = g & 3
        return [
            pltpu.make_async_copy(
                x_hbm.at[img, :, pl.ds(tr * 8, 8), pl.ds(tc * 128, 128)],
                buf,
                sem,
            )
        ]

    def out_copy(g, buf, sem):
        tr = base + (g >> 2)
        tc = g & 3
        return [
            pltpu.make_async_copy(
                buf,
                out_hbm.at[img, :, pl.ds(tr * 8, 8), pl.ds(tc * 128, 128)],
                sem,
            )
        ]

    def compute(xin, xout):
        @plsc.parallel_loop(0, _CHUNK, step=16, unroll=2)
        def _(o):
            _interp_group(luts, xin, xout, o)

    for d in in_copy(0, xin0, si0):
        d.start()
    for d in in_copy(1, xin1, si1):
        d.start()
    for d in lut_copies:
        d.wait()

    def pair(i, carry):
        g0 = 2 * i
        for buf_i, (g, xin, xout, si, so) in enumerate(
            ((g0, xin0, xout0, si0, so0), (g0 + 1, xin1, xout1, si1, so1))
        ):
            for d in in_copy(g, xin, si):
                d.wait()

            @pl.when(i > 0)
            def _():
                for d in out_copy(g, xout, so):
                    d.wait()

            compute(xin, xout)
            for d in out_copy(g, xout, so):
                d.start()

            @pl.when(g + 2 < _NCHUNK)
            def _():
                for d in in_copy(g + 2, xin, si):
                    d.start()

        return carry

    lax.fori_loop(0, _NCHUNK // 2, pair, 0)
    for d in out_copy(_NCHUNK - 2, xout0, so0):
        d.wait()
    for d in out_copy(_NCHUNK - 1, xout1, so1):
        d.wait()


@functools.cache
def _build():
    mesh = plsc.VectorSubcoreMesh(
        core_axis_name="c", subcore_axis_name="s", num_cores=2, num_subcores=16
    )
    return pl.kernel(
        _body,
        out_type=jax.ShapeDtypeStruct((_B, 3, 512, 512), jnp.float32),
        mesh=mesh,
        scratch_types=[
            pltpu.VMEM((_CSTRIDE,), jnp.float32),
            pltpu.VMEM((_CSTRIDE,), jnp.float32),
            pltpu.VMEM((_CSTRIDE,), jnp.float32),
            pltpu.VMEM((3, 8, 128), jnp.float32),
            pltpu.VMEM((3, 8, 128), jnp.float32),
            pltpu.VMEM((3, 8, 128), jnp.float32),
            pltpu.VMEM((3, 8, 128), jnp.float32),
            pltpu.SemaphoreType.DMA,
            pltpu.SemaphoreType.DMA,
            pltpu.SemaphoreType.DMA,
            pltpu.SemaphoreType.DMA,
        ],
        compiler_params=pltpu.CompilerParams(
            needs_layout_passes=False, use_tc_tiling_on_sc=True
        ),
    )


@jax.jit
def kernel(lut, x):
    lut_pad = (
        jnp.zeros((3, _CPAD), jnp.float32)
        .at[:, :_CSTRIDE]
        .set(lut.reshape(3, _CSTRIDE))
        .reshape(3 * _CPAD)
    )
    return _build()(lut_pad, x)


# speedup vs baseline: 1.1039x; 1.1039x over previous
import functools

import jax
import jax.numpy as jnp
from jax import lax
from jax.experimental import pallas as pl
from jax.experimental.pallas import tpu as pltpu
from jax.experimental.pallas import tpu_sc as plsc

_DIM = 33
_CSTRIDE = _DIM ** 3
_CPAD = 36096
_PLANE = 512 * 512
_B = 16
_PPW = _B * _PLANE // 32
_CHUNK = 1024
_NCHUNK = _PPW // _CHUNK
_GROUPS = _CHUNK // 16
_SCALE = float(_DIM - 1)


def _interp_group(luts, xin, xout, o):
    rr = o >> 7
    col = o & 127
    xr = xin[0, rr, pl.ds(col, 16)]
    xg = xin[1, rr, pl.ds(col, 16)]
    xb = xin[2, rr, pl.ds(col, 16)]
    r = xr * _SCALE
    g = xg * _SCALE
    b = xb * _SCALE
    r0 = jnp.minimum(r, float(_DIM - 2)).astype(jnp.int32)
    g0 = jnp.minimum(g, float(_DIM - 2)).astype(jnp.int32)
    b0 = jnp.minimum(b, float(_DIM - 2)).astype(jnp.int32)
    fr = r - r0.astype(jnp.float32)
    fg = g - g0.astype(jnp.float32)
    fb = b - b0.astype(jnp.float32)
    base = b0 * (_DIM * _DIM) + g0 * _DIM + r0
    cg = 1.0 - fg
    cb = 1.0 - fb
    t00 = cb * cg
    t01 = cb * fg
    t10 = fb * cg
    t11 = fb * fg
    cr = 1.0 - fr
    acc = [None, None, None]
    for off, tw, fw in (
        (0, t00, cr), (1, t00, fr),
        (_DIM, t01, cr), (_DIM + 1, t01, fr),
        (_DIM * _DIM, t10, cr), (_DIM * _DIM + 1, t10, fr),
        (_DIM * _DIM + _DIM, t11, cr), (_DIM * _DIM + _DIM + 1, t11, fr),
    ):
        idx = base + off if off else base
        w = tw * fw
        for c in range(3):
            v = w * plsc.load_gather(luts[c], [idx])
            acc[c] = v if acc[c] is None else acc[c] + v
    for c in range(3):
        xout[c, rr, pl.ds(col, 16)] = acc[c]


def _body(lut_hbm, x_hbm, out_hbm, lutr, lutg, lutb, xin0, xin1, xout0, xout1,
          si0, si1, so0, so1):
    img = lax.axis_index("s")
    half = lax.axis_index("c")
    base = half * 32
    luts = (lutr, lutg, lutb)
    lut_copies = [
        pltpu.make_async_copy(
            lut_hbm.at[pl.ds(c * _CPAD, _CSTRIDE)], luts[c], so0
        )
        for c in range(3)
    ]
    for d in lut_copies:
        d.start()

    def in_copy(g, buf, sem):
        tr = base + (g >> 2)
        tc = g & 3
        return [
            pltpu.make_async_copy(
                x_hbm.at[img, :, pl.ds(tr * 8, 8), pl.ds(tc * 128, 128)],
                buf,
                sem,
            )
        ]

    def out_copy(g, buf, sem):
        tr = base + (g >> 2)
        tc = g & 3
        return [
            pltpu.make_async_copy(
                buf,
                out_hbm.at[img, :, pl.ds(tr * 8, 8), pl.ds(tc * 128, 128)],
                sem,
            )
        ]

    def compute(xin, xout):
        @plsc.parallel_loop(0, _CHUNK, step=16, unroll=2)
        def _(o):
            _interp_group(luts, xin, xout, o)

    for d in in_copy(0, xin0, si0):
        d.start()
    for d in in_copy(1, xin1, si1):
        d.start()
    for d in lut_copies:
        d.wait()

    def pair(i, carry):
        g0 = 2 * i
        for buf_i, (g, xin, xout, si, so) in enumerate(
            ((g0, xin0, xout0, si0, so0), (g0 + 1, xin1, xout1, si1, so1))
        ):
            for d in in_copy(g, xin, si):
                d.wait()

            @pl.when(i > 0)
            def _():
                for d in out_copy(g, xout, so):
                    d.wait()

            compute(xin, xout)
            for d in out_copy(g, xout, so):
                d.start()

            @pl.when(g + 2 < _NCHUNK)
            def _():
                for d in in_copy(g + 2, xin, si):
                    d.start()

        return carry

    lax.fori_loop(0, _NCHUNK // 2, pair, 0)
    for d in out_copy(_NCHUNK - 2, xout0, so0):
        d.wait()
    for d in out_copy(_NCHUNK - 1, xout1, so1):
        d.wait()


@functools.cache
def _build():
    mesh = plsc.VectorSubcoreMesh(
        core_axis_name="c", subcore_axis_name="s", num_cores=2, num_subcores=16
    )
    return pl.kernel(
        _body,
        out_type=jax.ShapeDtypeStruct((_B, 3, 512, 512), jnp.float32),
        mesh=mesh,
        scratch_types=[
            pltpu.VMEM((_CSTRIDE,), jnp.float32),
            pltpu.VMEM((_CSTRIDE,), jnp.float32),
            pltpu.VMEM((_CSTRIDE,), jnp.float32),
            pltpu.VMEM((3, 8, 128), jnp.float32),
            pltpu.VMEM((3, 8, 128), jnp.float32),
            pltpu.VMEM((3, 8, 128), jnp.float32),
            pltpu.VMEM((3, 8, 128), jnp.float32),
            pltpu.SemaphoreType.DMA,
            pltpu.SemaphoreType.DMA,
            pltpu.SemaphoreType.DMA,
            pltpu.SemaphoreType.DMA,
        ],
        compiler_params=pltpu.CompilerParams(
            needs_layout_passes=False, use_tc_tiling_on_sc=True
        ),
    )


@jax.jit
def kernel(lut, x):
    lut_pad = (
        jnp.zeros((3, _CPAD), jnp.float32)
        .at[:, :_CSTRIDE]
        .set(lut.reshape(3, _CSTRIDE))
        .reshape(3 * _CPAD)
    )
    return _build()(lut_pad, x)
